# baseline (device time: 10267 ns/iter reference)
import jax
import jax.numpy as jnp
from jax import lax
from jax.experimental import pallas as pl
from jax.experimental.pallas import tpu as pltpu

N_DEV = 8
N_CHUNK = 4


def kernel(x):
    m_per, n = x.shape
    chunk = m_per // N_CHUNK

    def body(x_ref, out_ref, buf_ref, comm_ref, send_sems, recv_sems, copy_sems):
        my_pos = lax.axis_index("i")

        copies = [
            pltpu.make_async_copy(
                x_ref.at[pl.ds(c * chunk, chunk), :],
                buf_ref.at[c % 2],
                copy_sems.at[c % 2],
            )
            for c in range(N_CHUNK)
        ]
        copies[0].start()

        barrier_sem = pltpu.get_barrier_semaphore()
        for d in range(1, N_DEV):
            peer = lax.rem(my_pos + d, N_DEV)
            pl.semaphore_signal(
                barrier_sem, inc=1,
                device_id=(peer,), device_id_type=pl.DeviceIdType.MESH,
            )
        pl.semaphore_wait(barrier_sem, N_DEV - 1)

        partial = None
        for c in range(N_CHUNK):
            if c + 1 < N_CHUNK:
                copies[c + 1].start()
            copies[c].wait()
            s = jnp.sum(buf_ref[c % 2, :, :], axis=0, keepdims=True)
            partial = s if partial is None else partial + s
        comm_ref[pl.ds(0, 1), :] = partial

        rdmas = []
        for d in range(1, N_DEV):
            peer = lax.rem(my_pos + d, N_DEV)
            rdma = pltpu.make_async_remote_copy(
                src_ref=comm_ref.at[pl.ds(0, 1), :],
                dst_ref=comm_ref.at[pl.ds(d, 1), :],
                send_sem=send_sems.at[d],
                recv_sem=recv_sems.at[d],
                device_id=(peer,),
                device_id_type=pl.DeviceIdType.MESH,
            )
            rdma.start()
            rdmas.append(rdma)
        for rdma in rdmas:
            rdma.wait_recv()

        out_ref[:, :] = jnp.sum(comm_ref[:, :], axis=0, keepdims=True)

        for rdma in rdmas:
            rdma.wait_send()

    return pl.pallas_call(
        body,
        out_shape=jax.ShapeDtypeStruct((1, n), jnp.float32),
        in_specs=[pl.BlockSpec(memory_space=pl.ANY)],
        out_specs=pl.BlockSpec(memory_space=pltpu.VMEM),
        scratch_shapes=[
            pltpu.VMEM((2, chunk, n), jnp.float32),
            pltpu.VMEM((N_DEV, n), jnp.float32),
            pltpu.SemaphoreType.DMA((N_DEV,)),
            pltpu.SemaphoreType.DMA((N_DEV,)),
            pltpu.SemaphoreType.DMA((2,)),
        ],
        compiler_params=pltpu.CompilerParams(collective_id=0),
    )(x)
